# baseline (speedup 1.0000x reference)
import functools

import jax
import jax.numpy as jnp
from jax import lax
from jax.experimental import pallas as pl
from jax.experimental.pallas import tpu as pltpu


def _stage_conv(xt, w_ref, pad_ref, col_ref, *, H, W, Cin, Cout):
    pad_ref[0:1, :, :] = jnp.zeros((1, W + 2, Cin), jnp.bfloat16)
    pad_ref[H + 1:H + 2, :, :] = jnp.zeros((1, W + 2, Cin), jnp.bfloat16)
    pad_ref[:, 0:1, :] = jnp.zeros((H + 2, 1, Cin), jnp.bfloat16)
    pad_ref[:, W + 1:W + 2, :] = jnp.zeros((H + 2, 1, Cin), jnp.bfloat16)
    pad_ref[1:H + 1, 1:W + 1, :] = xt

    for g in range(3):
        col_ref[:, :, g * Cin:(g + 1) * Cin] = pad_ref[g:g + H, :, :]

    z = jnp.dot(col_ref[...].reshape(H * (W + 2), 3 * Cin), w_ref[...],
                preferred_element_type=jnp.float32)
    z = z.reshape(H, W + 2, 3 * Cout)

    return (z[:, 0:W, 0:Cout]
            + z[:, 1:W + 1, Cout:2 * Cout]
            + z[:, 2:W + 2, 2 * Cout:3 * Cout])


def _conv1_kernel(x_ref, w_ref, y_ref, stats_ref, pad0, col0, pad1, col1,
                  *, H, W, Cin, Cout, B):
    for i in range(B):
        pad_ref, col_ref = (pad0, col0) if i % 2 == 0 else (pad1, col1)
        xt = x_ref[i].astype(jnp.bfloat16).reshape(H, W, Cin)
        y = _stage_conv(xt, w_ref, pad_ref, col_ref,
                        H=H, W=W, Cin=Cin, Cout=Cout)
        stats_ref[i, 0:1, :] = jnp.sum(y, axis=(0, 1)).reshape(1, Cout)
        stats_ref[i, 1:2, :] = jnp.sum(y * y, axis=(0, 1)).reshape(1, Cout)
        y_ref[i] = y.astype(jnp.bfloat16)


def _conv2_kernel(x_ref, w_ref, scale_ref, shift_ref, y_ref, stats_ref,
                  pad0, col0, pad1, col1, *, H, W, Cin, Cout, B):
    for i in range(B):
        pad_ref, col_ref = (pad0, col0) if i % 2 == 0 else (pad1, col1)
        xa = x_ref[i].astype(jnp.float32) * scale_ref[...] + shift_ref[...]
        xt = jnp.maximum(xa, 0.0).astype(jnp.bfloat16)
        y = _stage_conv(xt, w_ref, pad_ref, col_ref,
                        H=H, W=W, Cin=Cin, Cout=Cout)
        stats_ref[i, 0:1, :] = jnp.sum(y, axis=(0, 1)).reshape(1, Cout)
        stats_ref[i, 1:2, :] = jnp.sum(y * y, axis=(0, 1)).reshape(1, Cout)
        y_ref[i] = y.astype(jnp.bfloat16)


def _final_kernel(y_ref, scale_ref, shift_ref, o_ref, *, B):
    for i in range(B):
        y = y_ref[i].astype(jnp.float32)
        o_ref[i] = jnp.maximum(y * scale_ref[...] + shift_ref[...], 0.0)


def _conv_scratch(H, W, Cin):
    return [
        pltpu.VMEM((H + 2, W + 2, Cin), jnp.bfloat16),
        pltpu.VMEM((H, W + 2, 3 * Cin), jnp.bfloat16),
        pltpu.VMEM((H + 2, W + 2, Cin), jnp.bfloat16),
        pltpu.VMEM((H, W + 2, 3 * Cin), jnp.bfloat16),
    ]


def _conv1(x, w_slab, *, H, W, Cin, Cout, B):
    N = x.shape[0]
    HW = H * W
    kern = functools.partial(_conv1_kernel, H=H, W=W, Cin=Cin, Cout=Cout, B=B)
    return pl.pallas_call(
        kern,
        grid=(N // B,),
        in_specs=[
            pl.BlockSpec((B, HW, Cin), lambda n: (n, 0, 0)),
            pl.BlockSpec((3 * Cin, 3 * Cout), lambda n: (0, 0)),
        ],
        out_specs=(
            pl.BlockSpec((B, H, W, Cout), lambda n: (n, 0, 0, 0)),
            pl.BlockSpec((B, 8, Cout), lambda n: (n, 0, 0)),
        ),
        out_shape=(
            jax.ShapeDtypeStruct((N, H, W, Cout), jnp.bfloat16),
            jax.ShapeDtypeStruct((N, 8, Cout), jnp.float32),
        ),
        scratch_shapes=_conv_scratch(H, W, Cin),
        compiler_params=pltpu.CompilerParams(
            dimension_semantics=("parallel",)),
    )(x, w_slab)


def _conv2(x, w_slab, scale, shift, *, H, W, Cin, Cout, B):
    N = x.shape[0]
    kern = functools.partial(_conv2_kernel, H=H, W=W, Cin=Cin, Cout=Cout, B=B)
    return pl.pallas_call(
        kern,
        grid=(N // B,),
        in_specs=[
            pl.BlockSpec((B, H, W, Cin), lambda n: (n, 0, 0, 0)),
            pl.BlockSpec((3 * Cin, 3 * Cout), lambda n: (0, 0)),
            pl.BlockSpec((1, Cin), lambda n: (0, 0)),
            pl.BlockSpec((1, Cin), lambda n: (0, 0)),
        ],
        out_specs=(
            pl.BlockSpec((B, H, W, Cout), lambda n: (n, 0, 0, 0)),
            pl.BlockSpec((B, 8, Cout), lambda n: (n, 0, 0)),
        ),
        out_shape=(
            jax.ShapeDtypeStruct((N, H, W, Cout), jnp.bfloat16),
            jax.ShapeDtypeStruct((N, 8, Cout), jnp.float32),
        ),
        scratch_shapes=_conv_scratch(H, W, Cin),
        compiler_params=pltpu.CompilerParams(
            dimension_semantics=("parallel",)),
    )(x, w_slab, scale, shift)


def _final(y, scale, shift, *, B):
    N, H, W, Cout = y.shape
    kern = functools.partial(_final_kernel, B=B)
    return pl.pallas_call(
        kern,
        grid=(N // B,),
        in_specs=[
            pl.BlockSpec((B, H, W, Cout), lambda n: (n, 0, 0, 0)),
            pl.BlockSpec((1, Cout), lambda n: (0, 0)),
            pl.BlockSpec((1, Cout), lambda n: (0, 0)),
        ],
        out_specs=pl.BlockSpec((B, H, W, Cout), lambda n: (n, 0, 0, 0)),
        out_shape=jax.ShapeDtypeStruct((N, H, W, Cout), jnp.float32),
        compiler_params=pltpu.CompilerParams(
            dimension_semantics=("parallel",)),
    )(y, scale, shift)


def _bn_affine(stats, gamma, beta, count, eps=1e-5):
    s = jnp.sum(stats[:, 0, :], axis=0)
    ss = jnp.sum(stats[:, 1, :], axis=0)
    mean = s / count
    var = jnp.maximum(ss / count - mean * mean, 0.0)
    scale = gamma * lax.rsqrt(var + eps)
    shift = beta - mean * scale
    cp = scale.shape[0]
    return scale.reshape(1, cp), shift.reshape(1, cp)


def kernel(x_nchw, w1, cb1, g1, b1, w2, cb2, g2, b2):
    del cb1, cb2
    N, Cin, H, W = x_nchw.shape
    Cout = w1.shape[0]
    HW = H * W
    count = float(N * HW)

    def slab(w):
        wt = jnp.transpose(w, (2, 1, 3, 0))
        return wt.reshape(3 * w.shape[1], 3 * w.shape[0]).astype(jnp.bfloat16)

    x_rows = jnp.transpose(x_nchw, (0, 2, 3, 1)).reshape(N, HW, Cin)

    B = 4 if N % 4 == 0 else 1
    y1, st1 = _conv1(x_rows, slab(w1), H=H, W=W, Cin=Cin, Cout=Cout, B=B)
    scale1, shift1 = _bn_affine(st1, g1, b1, count)
    y2, st2 = _conv2(y1, slab(w2), scale1, shift1,
                     H=H, W=W, Cin=Cout, Cout=Cout, B=B)
    scale2, shift2 = _bn_affine(st2, g2, b2, count)
    out = _final(y2, scale2, shift2, B=B)

    return jnp.transpose(out, (0, 3, 1, 2))


# speedup vs baseline: 1.4273x; 1.4273x over previous
import functools

import jax
import jax.numpy as jnp
from jax import lax
from jax.experimental import pallas as pl
from jax.experimental.pallas import tpu as pltpu


def _stage_conv(xt, w_ref, pad_ref, col_ref, *, H, W, Cin):
    pad_ref[0:1, :, :] = jnp.zeros((1, W + 2, Cin), jnp.bfloat16)
    pad_ref[H + 1:H + 2, :, :] = jnp.zeros((1, W + 2, Cin), jnp.bfloat16)
    pad_ref[:, 0:1, :] = jnp.zeros((H + 2, 1, Cin), jnp.bfloat16)
    pad_ref[:, W + 1:W + 2, :] = jnp.zeros((H + 2, 1, Cin), jnp.bfloat16)
    pad_ref[1:H + 1, 1:W + 1, :] = xt.reshape(H, W, Cin)

    for t in range(9):
        dy, dx = divmod(t, 3)
        col_ref[:, :, t * Cin:(t + 1) * Cin] = pad_ref[dy:dy + H, dx:dx + W, :]

    return jnp.dot(col_ref[...].reshape(H * W, 9 * Cin), w_ref[...],
                   preferred_element_type=jnp.float32)


def _conv1_kernel(x_ref, w_ref, y_ref, stats_ref, pad0, col0, pad1, col1,
                  *, H, W, Cin, B):
    for i in range(B):
        pad_ref, col_ref = (pad0, col0) if i % 2 == 0 else (pad1, col1)
        xt = x_ref[i]
        y = _stage_conv(xt, w_ref, pad_ref, col_ref, H=H, W=W, Cin=Cin)
        stats_ref[i, 0:1, :] = jnp.sum(y, axis=0, keepdims=True)
        stats_ref[i, 1:2, :] = jnp.sum(y * y, axis=0, keepdims=True)
        y_ref[i] = y.astype(jnp.bfloat16)


def _conv2_kernel(x_ref, w_ref, scale_ref, shift_ref, y_ref, stats_ref,
                  pad0, col0, pad1, col1, *, H, W, Cin, B):
    for i in range(B):
        pad_ref, col_ref = (pad0, col0) if i % 2 == 0 else (pad1, col1)
        xa = x_ref[i].astype(jnp.float32) * scale_ref[...] + shift_ref[...]
        xt = jnp.maximum(xa, 0.0).astype(jnp.bfloat16)
        y = _stage_conv(xt, w_ref, pad_ref, col_ref, H=H, W=W, Cin=Cin)
        stats_ref[i, 0:1, :] = jnp.sum(y, axis=0, keepdims=True)
        stats_ref[i, 1:2, :] = jnp.sum(y * y, axis=0, keepdims=True)
        y_ref[i] = y.astype(jnp.bfloat16)


def _final_kernel(y_ref, scale_ref, shift_ref, o_ref, *, B):
    for i in range(B):
        y = y_ref[i].astype(jnp.float32)
        o_ref[i] = jnp.maximum(y * scale_ref[...] + shift_ref[...], 0.0)


def _conv_scratch(H, W, Cin):
    return [
        pltpu.VMEM((H + 2, W + 2, Cin), jnp.bfloat16),
        pltpu.VMEM((H, W, 9 * Cin), jnp.bfloat16),
        pltpu.VMEM((H + 2, W + 2, Cin), jnp.bfloat16),
        pltpu.VMEM((H, W, 9 * Cin), jnp.bfloat16),
    ]


def _conv1(x, w_slab, *, H, W, Cin, Cout, B):
    N = x.shape[0]
    HW = H * W
    kern = functools.partial(_conv1_kernel, H=H, W=W, Cin=Cin, B=B)
    return pl.pallas_call(
        kern,
        grid=(N // B,),
        in_specs=[
            pl.BlockSpec((B, HW, Cin), lambda n: (n, 0, 0)),
            pl.BlockSpec((9 * Cin, Cout), lambda n: (0, 0)),
        ],
        out_specs=(
            pl.BlockSpec((B, HW, Cout), lambda n: (n, 0, 0)),
            pl.BlockSpec((B, 8, Cout), lambda n: (n, 0, 0)),
        ),
        out_shape=(
            jax.ShapeDtypeStruct((N, HW, Cout), jnp.bfloat16),
            jax.ShapeDtypeStruct((N, 8, Cout), jnp.float32),
        ),
        scratch_shapes=_conv_scratch(H, W, Cin),
        compiler_params=pltpu.CompilerParams(
            dimension_semantics=("parallel",)),
    )(x, w_slab)


def _conv2(x, w_slab, scale, shift, *, H, W, Cin, Cout, B):
    N = x.shape[0]
    HW = H * W
    kern = functools.partial(_conv2_kernel, H=H, W=W, Cin=Cin, B=B)
    return pl.pallas_call(
        kern,
        grid=(N // B,),
        in_specs=[
            pl.BlockSpec((B, HW, Cin), lambda n: (n, 0, 0)),
            pl.BlockSpec((9 * Cin, Cout), lambda n: (0, 0)),
            pl.BlockSpec((1, Cin), lambda n: (0, 0)),
            pl.BlockSpec((1, Cin), lambda n: (0, 0)),
        ],
        out_specs=(
            pl.BlockSpec((B, HW, Cout), lambda n: (n, 0, 0)),
            pl.BlockSpec((B, 8, Cout), lambda n: (n, 0, 0)),
        ),
        out_shape=(
            jax.ShapeDtypeStruct((N, HW, Cout), jnp.bfloat16),
            jax.ShapeDtypeStruct((N, 8, Cout), jnp.float32),
        ),
        scratch_shapes=_conv_scratch(H, W, Cin),
        compiler_params=pltpu.CompilerParams(
            dimension_semantics=("parallel",)),
    )(x, w_slab, scale, shift)


def _final(y, scale, shift, *, B):
    N, HW, Cout = y.shape
    kern = functools.partial(_final_kernel, B=B)
    return pl.pallas_call(
        kern,
        grid=(N // B,),
        in_specs=[
            pl.BlockSpec((B, HW, Cout), lambda n: (n, 0, 0)),
            pl.BlockSpec((1, Cout), lambda n: (0, 0)),
            pl.BlockSpec((1, Cout), lambda n: (0, 0)),
        ],
        out_specs=pl.BlockSpec((B, HW, Cout), lambda n: (n, 0, 0)),
        out_shape=jax.ShapeDtypeStruct((N, HW, Cout), jnp.float32),
        compiler_params=pltpu.CompilerParams(
            dimension_semantics=("parallel",)),
    )(y, scale, shift)


def _bn_affine(stats, gamma, beta, count, eps=1e-5):
    s = jnp.sum(stats[:, 0, :], axis=0)
    ss = jnp.sum(stats[:, 1, :], axis=0)
    mean = s / count
    var = jnp.maximum(ss / count - mean * mean, 0.0)
    scale = gamma * lax.rsqrt(var + eps)
    shift = beta - mean * scale
    cp = scale.shape[0]
    return scale.reshape(1, cp), shift.reshape(1, cp)


def kernel(x_nchw, w1, cb1, g1, b1, w2, cb2, g2, b2):
    del cb1, cb2
    N, Cin, H, W = x_nchw.shape
    Cout = w1.shape[0]
    HW = H * W
    count = float(N * HW)

    def slab(w):
        wt = jnp.transpose(w, (2, 3, 1, 0))
        return wt.reshape(9 * w.shape[1], w.shape[0]).astype(jnp.bfloat16)

    x_rows = jnp.transpose(x_nchw, (0, 2, 3, 1)).reshape(N, HW, Cin)
    x_rows = x_rows.astype(jnp.bfloat16)

    B = 4 if N % 4 == 0 else 1
    y1, st1 = _conv1(x_rows, slab(w1), H=H, W=W, Cin=Cin, Cout=Cout, B=B)
    scale1, shift1 = _bn_affine(st1, g1, b1, count)
    y2, st2 = _conv2(y1, slab(w2), scale1, shift1,
                     H=H, W=W, Cin=Cout, Cout=Cout, B=B)
    scale2, shift2 = _bn_affine(st2, g2, b2, count)
    out = _final(y2, scale2, shift2, B=B)

    out = out.reshape(N, H, W, Cout)
    return jnp.transpose(out, (0, 3, 1, 2))


# speedup vs baseline: 1.5350x; 1.0755x over previous
import functools

import jax
import jax.numpy as jnp
from jax import lax
from jax.experimental import pallas as pl
from jax.experimental.pallas import tpu as pltpu


def _stage_conv(xt, w_ref, pad_ref, col_ref, *, H, W, Cin):
    pad_ref[0:1, :, :] = jnp.zeros((1, W + 2, Cin), jnp.bfloat16)
    pad_ref[H + 1:H + 2, :, :] = jnp.zeros((1, W + 2, Cin), jnp.bfloat16)
    pad_ref[:, 0:1, :] = jnp.zeros((H + 2, 1, Cin), jnp.bfloat16)
    pad_ref[:, W + 1:W + 2, :] = jnp.zeros((H + 2, 1, Cin), jnp.bfloat16)
    pad_ref[1:H + 1, 1:W + 1, :] = xt.reshape(H, W, Cin)

    for t in range(9):
        dy, dx = divmod(t, 3)
        col_ref[:, :, t * Cin:(t + 1) * Cin] = pad_ref[dy:dy + H, dx:dx + W, :]

    return jnp.dot(col_ref[...].reshape(H * W, 9 * Cin), w_ref[...],
                   preferred_element_type=jnp.float32)


def _conv1_kernel(x_ref, w_ref, y_ref, stats_ref, pad0, col0, pad1, col1,
                  *, H, W, Cin, B):
    @pl.when(pl.program_id(0) == 0)
    def _init():
        stats_ref[...] = jnp.zeros_like(stats_ref)

    s_tot = 0.0
    ss_tot = 0.0
    for i in range(B):
        pad_ref, col_ref = (pad0, col0) if i % 2 == 0 else (pad1, col1)
        xt = x_ref[i].astype(jnp.bfloat16)
        y = _stage_conv(xt, w_ref, pad_ref, col_ref, H=H, W=W, Cin=Cin)
        s_tot = s_tot + jnp.sum(y, axis=0, keepdims=True)
        ss_tot = ss_tot + jnp.sum(y * y, axis=0, keepdims=True)
        y_ref[i] = y.astype(jnp.bfloat16)
    stats_ref[0:1, :] += s_tot
    stats_ref[1:2, :] += ss_tot


def _conv2_kernel(x_ref, w_ref, scale_ref, shift_ref, y_ref, stats_ref,
                  pad0, col0, pad1, col1, *, H, W, Cin, B):
    @pl.when(pl.program_id(0) == 0)
    def _init():
        stats_ref[...] = jnp.zeros_like(stats_ref)

    s_tot = 0.0
    ss_tot = 0.0
    for i in range(B):
        pad_ref, col_ref = (pad0, col0) if i % 2 == 0 else (pad1, col1)
        xa = x_ref[i].astype(jnp.float32) * scale_ref[...] + shift_ref[...]
        xt = jnp.maximum(xa, 0.0).astype(jnp.bfloat16)
        y = _stage_conv(xt, w_ref, pad_ref, col_ref, H=H, W=W, Cin=Cin)
        s_tot = s_tot + jnp.sum(y, axis=0, keepdims=True)
        ss_tot = ss_tot + jnp.sum(y * y, axis=0, keepdims=True)
        y_ref[i] = y.astype(jnp.bfloat16)
    stats_ref[0:1, :] += s_tot
    stats_ref[1:2, :] += ss_tot


def _final_kernel(y_ref, scale_ref, shift_ref, o_ref, *, B):
    for i in range(B):
        y = y_ref[i].astype(jnp.float32)
        o_ref[i] = jnp.maximum(y * scale_ref[...] + shift_ref[...], 0.0)


def _conv_scratch(H, W, Cin):
    return [
        pltpu.VMEM((H + 2, W + 2, Cin), jnp.bfloat16),
        pltpu.VMEM((H, W, 9 * Cin), jnp.bfloat16),
        pltpu.VMEM((H + 2, W + 2, Cin), jnp.bfloat16),
        pltpu.VMEM((H, W, 9 * Cin), jnp.bfloat16),
    ]


def _conv1(x, w_slab, *, H, W, Cin, Cout, B):
    N = x.shape[0]
    HW = H * W
    kern = functools.partial(_conv1_kernel, H=H, W=W, Cin=Cin, B=B)
    return pl.pallas_call(
        kern,
        grid=(N // B,),
        in_specs=[
            pl.BlockSpec((B, HW, Cin), lambda n: (n, 0, 0)),
            pl.BlockSpec((9 * Cin, Cout), lambda n: (0, 0)),
        ],
        out_specs=(
            pl.BlockSpec((B, HW, Cout), lambda n: (n, 0, 0)),
            pl.BlockSpec((8, Cout), lambda n: (0, 0)),
        ),
        out_shape=(
            jax.ShapeDtypeStruct((N, HW, Cout), jnp.bfloat16),
            jax.ShapeDtypeStruct((8, Cout), jnp.float32),
        ),
        scratch_shapes=_conv_scratch(H, W, Cin),
        compiler_params=pltpu.CompilerParams(
            dimension_semantics=("arbitrary",)),
    )(x, w_slab)


def _conv2(x, w_slab, scale, shift, *, H, W, Cin, Cout, B):
    N = x.shape[0]
    HW = H * W
    kern = functools.partial(_conv2_kernel, H=H, W=W, Cin=Cin, B=B)
    return pl.pallas_call(
        kern,
        grid=(N // B,),
        in_specs=[
            pl.BlockSpec((B, HW, Cin), lambda n: (n, 0, 0)),
            pl.BlockSpec((9 * Cin, Cout), lambda n: (0, 0)),
            pl.BlockSpec((1, Cin), lambda n: (0, 0)),
            pl.BlockSpec((1, Cin), lambda n: (0, 0)),
        ],
        out_specs=(
            pl.BlockSpec((B, HW, Cout), lambda n: (n, 0, 0)),
            pl.BlockSpec((8, Cout), lambda n: (0, 0)),
        ),
        out_shape=(
            jax.ShapeDtypeStruct((N, HW, Cout), jnp.bfloat16),
            jax.ShapeDtypeStruct((8, Cout), jnp.float32),
        ),
        scratch_shapes=_conv_scratch(H, W, Cin),
        compiler_params=pltpu.CompilerParams(
            dimension_semantics=("arbitrary",)),
    )(x, w_slab, scale, shift)


def _final(y, scale, shift, *, B):
    N, HW, Cout = y.shape
    kern = functools.partial(_final_kernel, B=B)
    return pl.pallas_call(
        kern,
        grid=(N // B,),
        in_specs=[
            pl.BlockSpec((B, HW, Cout), lambda n: (n, 0, 0)),
            pl.BlockSpec((1, Cout), lambda n: (0, 0)),
            pl.BlockSpec((1, Cout), lambda n: (0, 0)),
        ],
        out_specs=pl.BlockSpec((B, HW, Cout), lambda n: (n, 0, 0)),
        out_shape=jax.ShapeDtypeStruct((N, HW, Cout), jnp.float32),
        compiler_params=pltpu.CompilerParams(
            dimension_semantics=("parallel",)),
    )(y, scale, shift)


def _bn_affine(stats, gamma, beta, count, eps=1e-5):
    s = stats[0, :]
    ss = stats[1, :]
    mean = s / count
    var = jnp.maximum(ss / count - mean * mean, 0.0)
    scale = gamma * lax.rsqrt(var + eps)
    shift = beta - mean * scale
    cp = scale.shape[0]
    return scale.reshape(1, cp), shift.reshape(1, cp)


def kernel(x_nchw, w1, cb1, g1, b1, w2, cb2, g2, b2):
    del cb1, cb2
    N, Cin, H, W = x_nchw.shape
    Cout = w1.shape[0]
    HW = H * W
    count = float(N * HW)

    def slab(w):
        wt = jnp.transpose(w, (2, 3, 1, 0))
        return wt.reshape(9 * w.shape[1], w.shape[0]).astype(jnp.bfloat16)

    x_rows = jnp.transpose(x_nchw, (0, 2, 3, 1)).reshape(N, HW, Cin)

    B = 4 if N % 4 == 0 else 1
    y1, st1 = _conv1(x_rows, slab(w1), H=H, W=W, Cin=Cin, Cout=Cout, B=B)
    scale1, shift1 = _bn_affine(st1, g1, b1, count)
    y2, st2 = _conv2(y1, slab(w2), scale1, shift1,
                     H=H, W=W, Cin=Cout, Cout=Cout, B=B)
    scale2, shift2 = _bn_affine(st2, g2, b2, count)
    out = _final(y2, scale2, shift2, B=B)

    out = out.reshape(N, H, W, Cout)
    return jnp.transpose(out, (0, 3, 1, 2))


# speedup vs baseline: 1.5905x; 1.0361x over previous
import functools

import jax
import jax.numpy as jnp
from jax import lax
from jax.experimental import pallas as pl
from jax.experimental.pallas import tpu as pltpu


def _stage_conv(xt, w_ref, pad_ref, col_ref, *, H, W, Cin):
    pad_ref[0:1, :, :] = jnp.zeros((1, W + 2, Cin), jnp.bfloat16)
    pad_ref[H + 1:H + 2, :, :] = jnp.zeros((1, W + 2, Cin), jnp.bfloat16)
    pad_ref[:, 0:1, :] = jnp.zeros((H + 2, 1, Cin), jnp.bfloat16)
    pad_ref[:, W + 1:W + 2, :] = jnp.zeros((H + 2, 1, Cin), jnp.bfloat16)
    pad_ref[1:H + 1, 1:W + 1, :] = xt.reshape(H, W, Cin)

    for t in range(9):
        dy, dx = divmod(t, 3)
        col_ref[:, :, t * Cin:(t + 1) * Cin] = pad_ref[dy:dy + H, dx:dx + W, :]

    return jnp.dot(col_ref[...].reshape(H * W, 9 * Cin), w_ref[...],
                   preferred_element_type=jnp.float32)


def _conv1_kernel(x_ref, w_ref, y_ref, stats_ref, pad0, col0, pad1, col1,
                  *, H, W, Cin, B):
    @pl.when(pl.program_id(0) == 0)
    def _init():
        stats_ref[...] = jnp.zeros_like(stats_ref)

    s_tot = 0.0
    ss_tot = 0.0
    for i in range(B):
        pad_ref, col_ref = (pad0, col0) if i % 2 == 0 else (pad1, col1)
        xt = x_ref[i].astype(jnp.bfloat16)
        y = _stage_conv(xt, w_ref, pad_ref, col_ref, H=H, W=W, Cin=Cin)
        s_tot = s_tot + jnp.sum(y, axis=0, keepdims=True)
        ss_tot = ss_tot + jnp.sum(y * y, axis=0, keepdims=True)
        y_ref[i] = y.astype(jnp.bfloat16)
    stats_ref[0:1, :] += s_tot
    stats_ref[1:2, :] += ss_tot


def _conv2_kernel(x_ref, w_ref, scale_ref, shift_ref, y_ref, stats_ref,
                  pad0, col0, pad1, col1, *, H, W, Cin, B):
    @pl.when(pl.program_id(0) == 0)
    def _init():
        stats_ref[...] = jnp.zeros_like(stats_ref)

    s_tot = 0.0
    ss_tot = 0.0
    for i in range(B):
        pad_ref, col_ref = (pad0, col0) if i % 2 == 0 else (pad1, col1)
        xa = x_ref[i].astype(jnp.float32) * scale_ref[...] + shift_ref[...]
        xt = jnp.maximum(xa, 0.0).astype(jnp.bfloat16)
        y = _stage_conv(xt, w_ref, pad_ref, col_ref, H=H, W=W, Cin=Cin)
        s_tot = s_tot + jnp.sum(y, axis=0, keepdims=True)
        ss_tot = ss_tot + jnp.sum(y * y, axis=0, keepdims=True)
        y_ref[i] = y.astype(jnp.bfloat16)
    stats_ref[0:1, :] += s_tot
    stats_ref[1:2, :] += ss_tot


def _final_kernel(y_ref, scale_ref, shift_ref, o_ref, *, B):
    for i in range(B):
        y = y_ref[i].astype(jnp.float32)
        o_ref[i] = jnp.maximum(y * scale_ref[...] + shift_ref[...], 0.0)


def _conv_scratch(H, W, Cin):
    return [
        pltpu.VMEM((H + 2, W + 2, Cin), jnp.bfloat16),
        pltpu.VMEM((H, W, 9 * Cin), jnp.bfloat16),
        pltpu.VMEM((H + 2, W + 2, Cin), jnp.bfloat16),
        pltpu.VMEM((H, W, 9 * Cin), jnp.bfloat16),
    ]


def _conv1(x, w_slab, *, H, W, Cin, Cout, B):
    N = x.shape[0]
    HW = H * W
    kern = functools.partial(_conv1_kernel, H=H, W=W, Cin=Cin, B=B)
    return pl.pallas_call(
        kern,
        grid=(N // B,),
        in_specs=[
            pl.BlockSpec((B, HW, Cin), lambda n: (n, 0, 0)),
            pl.BlockSpec((9 * Cin, Cout), lambda n: (0, 0)),
        ],
        out_specs=(
            pl.BlockSpec((B, HW, Cout), lambda n: (n, 0, 0)),
            pl.BlockSpec((8, Cout), lambda n: (0, 0)),
        ),
        out_shape=(
            jax.ShapeDtypeStruct((N, HW, Cout), jnp.bfloat16),
            jax.ShapeDtypeStruct((8, Cout), jnp.float32),
        ),
        scratch_shapes=_conv_scratch(H, W, Cin),
        compiler_params=pltpu.CompilerParams(
            dimension_semantics=("arbitrary",)),
    )(x, w_slab)


def _conv2(x, w_slab, scale, shift, *, H, W, Cin, Cout, B):
    N = x.shape[0]
    HW = H * W
    kern = functools.partial(_conv2_kernel, H=H, W=W, Cin=Cin, B=B)
    return pl.pallas_call(
        kern,
        grid=(N // B,),
        in_specs=[
            pl.BlockSpec((B, HW, Cin), lambda n: (n, 0, 0)),
            pl.BlockSpec((9 * Cin, Cout), lambda n: (0, 0)),
            pl.BlockSpec((1, Cin), lambda n: (0, 0)),
            pl.BlockSpec((1, Cin), lambda n: (0, 0)),
        ],
        out_specs=(
            pl.BlockSpec((B, HW, Cout), lambda n: (n, 0, 0)),
            pl.BlockSpec((8, Cout), lambda n: (0, 0)),
        ),
        out_shape=(
            jax.ShapeDtypeStruct((N, HW, Cout), jnp.bfloat16),
            jax.ShapeDtypeStruct((8, Cout), jnp.float32),
        ),
        scratch_shapes=_conv_scratch(H, W, Cin),
        compiler_params=pltpu.CompilerParams(
            dimension_semantics=("arbitrary",)),
    )(x, w_slab, scale, shift)


def _final(y, scale, shift, *, B):
    N, HW, Cout = y.shape
    kern = functools.partial(_final_kernel, B=B)
    return pl.pallas_call(
        kern,
        grid=(N // B,),
        in_specs=[
            pl.BlockSpec((B, HW, Cout), lambda n: (n, 0, 0)),
            pl.BlockSpec((1, Cout), lambda n: (0, 0)),
            pl.BlockSpec((1, Cout), lambda n: (0, 0)),
        ],
        out_specs=pl.BlockSpec((B, HW, Cout), lambda n: (n, 0, 0)),
        out_shape=jax.ShapeDtypeStruct((N, HW, Cout), jnp.float32),
        compiler_params=pltpu.CompilerParams(
            dimension_semantics=("parallel",)),
    )(y, scale, shift)


def _bn_affine(stats, gamma, beta, count, eps=1e-5):
    s = stats[0, :]
    ss = stats[1, :]
    mean = s / count
    var = jnp.maximum(ss / count - mean * mean, 0.0)
    scale = gamma * lax.rsqrt(var + eps)
    shift = beta - mean * scale
    cp = scale.shape[0]
    return scale.reshape(1, cp), shift.reshape(1, cp)


def kernel(x_nchw, w1, cb1, g1, b1, w2, cb2, g2, b2):
    del cb1, cb2
    N, Cin, H, W = x_nchw.shape
    Cout = w1.shape[0]
    HW = H * W
    count = float(N * HW)

    def slab(w):
        wt = jnp.transpose(w, (2, 3, 1, 0))
        return wt.reshape(9 * w.shape[1], w.shape[0]).astype(jnp.bfloat16)

    x_rows = jnp.transpose(x_nchw, (0, 2, 3, 1)).reshape(N, HW, Cin)

    B = 8 if N % 8 == 0 else (4 if N % 4 == 0 else 1)
    y1, st1 = _conv1(x_rows, slab(w1), H=H, W=W, Cin=Cin, Cout=Cout, B=B)
    scale1, shift1 = _bn_affine(st1, g1, b1, count)
    y2, st2 = _conv2(y1, slab(w2), scale1, shift1,
                     H=H, W=W, Cin=Cout, Cout=Cout, B=B)
    scale2, shift2 = _bn_affine(st2, g2, b2, count)
    out = _final(y2, scale2, shift2, B=B)

    out = out.reshape(N, H, W, Cout)
    return jnp.transpose(out, (0, 3, 1, 2))
